# baseline (speedup 1.0000x reference)
import functools

import jax
import jax.numpy as jnp
from jax import lax
from jax.experimental import pallas as pl
from jax.experimental.pallas import tpu as pltpu
from jax.experimental.pallas import tpu_sc as plsc

_B = 16384
_F = 26
_V = 100000
_D = 64
_H1 = 256
_H2 = 128

_NC = 2
_NS = 16
_NW = _NC * _NS
_BPW = _B // _NW
_CB = 32
_NBLK = _BPW // _CB


def _sc_bi_interaction(x_t, tab128):
    mesh = plsc.VectorSubcoreMesh(core_axis_name="c", subcore_axis_name="s")

    HSL = 256
    NH = _BPW // HSL
    NBH = HSL // _CB

    @functools.partial(
        pl.kernel,
        out_type=jax.ShapeDtypeStruct((_B, _D), jnp.float32),
        mesh=mesh,
        scratch_types=[
            pltpu.VMEM((_F, HSL), jnp.int32),
            pltpu.VMEM((_F, _CB), jnp.int32),
            pltpu.VMEM((_F, _CB, 2 * _D), jnp.float32),
            pltpu.VMEM((_CB, _D), jnp.float32),
            pltpu.SemaphoreType.DMA,
        ],
        compiler_params=pltpu.CompilerParams(use_tc_tiling_on_sc=True,
                                             needs_layout_passes=False),
    )
    def sc_kernel(x_hbm, tab_hbm, out_hbm, xw, idxb, rows, bi, sem):
        wid = lax.axis_index("s") * _NC + lax.axis_index("c")
        base = wid * _BPW

        def half_body(h, _):
            h0 = base + h * HSL
            pltpu.sync_copy(x_hbm.at[:, pl.ds(h0, HSL)], xw)

            def block_body(blk, _):
                lo = blk * _CB
                b0 = h0 + lo
                for f in range(_F):
                    for j in range(_CB // 16):
                        idxb[f, pl.ds(j * 16, 16)] = (
                            xw[f, pl.ds(lo + j * 16, 16)] + f * _V)
                copies = [
                    pltpu.async_copy(tab_hbm.at[idxb.at[f]], rows.at[f], sem)
                    for f in range(_F)
                ]
                for c in copies:
                    c.wait()

                def elem_body(c, _):
                    for dd in range(_D // 16):
                        s = jnp.zeros((16,), jnp.float32)
                        ss = jnp.zeros((16,), jnp.float32)
                        for f in range(_F):
                            r = rows[f, c, pl.ds(dd * 16, 16)]
                            s = s + r
                            ss = ss + r * r
                        bi[c, pl.ds(dd * 16, 16)] = 0.5 * (s * s - ss)
                    return ()

                lax.fori_loop(0, _CB, elem_body, ())
                pltpu.sync_copy(bi, out_hbm.at[pl.ds(b0, _CB)])
                return ()

            lax.fori_loop(0, NBH, block_body, ())
            return ()

        lax.fori_loop(0, NH, half_body, ())

    return sc_kernel(x_t, tab128)


_VC = 512
_NVC = -(-_V // _VC)


def _tc_relayout(tab_t):
    def body(in_ref, out_ref):
        x = in_ref[0]
        xt = jnp.transpose(x)
        out_ref[0] = jnp.concatenate([xt, xt], axis=1)

    return pl.pallas_call(
        body,
        grid=(_F, _NVC),
        in_specs=[pl.BlockSpec((1, _D, _VC), lambda f, v: (f, 0, v))],
        out_specs=pl.BlockSpec((1, _VC, 2 * _D), lambda f, v: (f, v, 0)),
        out_shape=jax.ShapeDtypeStruct((_F, _V, 2 * _D), jnp.float32),
    )(tab_t)


def _tc_head(bi, x, W1, b1, W2, b2, W3t, b3, Wlt, bl):
    BT = 512
    grid = (_B // BT,)

    def body(bi_ref, x_ref, w1_ref, b1_ref, w2_ref, b2_ref, w3_ref,
             b3_ref, wl_ref, bl_ref, out_ref):
        h = jnp.dot(bi_ref[...], w1_ref[...],
                    preferred_element_type=jnp.float32) + b1_ref[...]
        h = jnp.maximum(h, 0.0)
        h = jnp.dot(h, w2_ref[...],
                    preferred_element_type=jnp.float32) + b2_ref[...]
        h = jnp.maximum(h, 0.0)
        deep = jnp.sum(h * w3_ref[...], axis=1, keepdims=True)
        xf = x_ref[...].astype(jnp.float32)
        lin = jnp.dot(xf, wl_ref[...])
        z = lin + deep + b3_ref[0, 0] + bl_ref[0, 0]
        out_ref[...] = jax.nn.sigmoid(z)

    return pl.pallas_call(
        body,
        grid=grid,
        in_specs=[
            pl.BlockSpec((BT, _D), lambda i: (i, 0)),
            pl.BlockSpec((BT, _F), lambda i: (i, 0)),
            pl.BlockSpec((_D, _H1), lambda i: (0, 0)),
            pl.BlockSpec((1, _H1), lambda i: (0, 0)),
            pl.BlockSpec((_H1, _H2), lambda i: (0, 0)),
            pl.BlockSpec((1, _H2), lambda i: (0, 0)),
            pl.BlockSpec((1, _H2), lambda i: (0, 0)),
            pl.BlockSpec((1, 1), lambda i: (0, 0)),
            pl.BlockSpec((_F, 1), lambda i: (0, 0)),
            pl.BlockSpec((1, 1), lambda i: (0, 0)),
        ],
        out_specs=pl.BlockSpec((BT, 1), lambda i: (i, 0)),
        out_shape=jax.ShapeDtypeStruct((_B, 1), jnp.float32),
    )(bi, x, W1, b1, W2, b2, W3t, b3, Wlt, bl)


def kernel(x, tables, Wl, bl, W1, b1, W2, b2, W3, b3):
    x = x.astype(jnp.int32)
    x_t = x.T
    tab_t = jnp.swapaxes(tables, 1, 2)
    tab128 = _tc_relayout(tab_t).reshape(_F * _V, 2 * _D)
    bi = _sc_bi_interaction(x_t, tab128)
    return _tc_head(
        bi, x, W1, b1.reshape(1, _H1), W2, b2.reshape(1, _H2),
        W3.reshape(1, _H2), b3.reshape(1, 1), Wl,
        bl.reshape(1, 1),
    )


# speedup vs baseline: 3.3148x; 3.3148x over previous
import functools

import jax
import jax.numpy as jnp
from jax import lax
from jax.experimental import pallas as pl
from jax.experimental.pallas import tpu as pltpu
from jax.experimental.pallas import tpu_sc as plsc

_B = 16384
_F = 26
_V = 100000
_D = 64
_H1 = 256
_H2 = 128

_NC = 2
_NS = 16
_NW = _NC * _NS
_BPW = _B // _NW
_CB = 32
_NBLK = _BPW // _CB


def _sc_bi_interaction(x_t, tab128):
    mesh = plsc.VectorSubcoreMesh(core_axis_name="c", subcore_axis_name="s")

    HSL = 256
    NH = _BPW // HSL
    NBH = HSL // _CB

    @functools.partial(
        pl.kernel,
        out_type=jax.ShapeDtypeStruct((_B, _D), jnp.float32),
        mesh=mesh,
        scratch_types=[
            pltpu.VMEM((_F, HSL), jnp.int32),
            pltpu.VMEM((_F, _CB), jnp.int32),
            pltpu.VMEM((_F, _CB, 2 * _D), jnp.float32),
            pltpu.VMEM((_CB, _D), jnp.float32),
            pltpu.SemaphoreType.DMA,
        ],
        compiler_params=pltpu.CompilerParams(use_tc_tiling_on_sc=True,
                                             needs_layout_passes=False),
    )
    def sc_kernel(x_hbm, tab_hbm, out_hbm, xw, idxb, rows, bi, sem):
        wid = lax.axis_index("s") * _NC + lax.axis_index("c")
        base = wid * _BPW

        def half_body(h, _):
            h0 = base + h * HSL
            pltpu.sync_copy(x_hbm.at[:, pl.ds(h0, HSL)], xw)

            def block_body(blk, _):
                lo = blk * _CB
                b0 = h0 + lo
                for f in range(_F):
                    for j in range(_CB // 16):
                        idxb[f, pl.ds(j * 16, 16)] = (
                            xw[f, pl.ds(lo + j * 16, 16)] + f * _V)
                copies = [
                    pltpu.async_copy(tab_hbm.at[idxb.at[f]], rows.at[f], sem)
                    for f in range(_F)
                ]
                for c in copies:
                    c.wait()

                def elem_body(c, _):
                    for dd in range(_D // 16):
                        s = jnp.zeros((16,), jnp.float32)
                        ss = jnp.zeros((16,), jnp.float32)
                        for f in range(_F):
                            r = rows[f, c, pl.ds(dd * 16, 16)]
                            s = s + r
                            ss = ss + r * r
                        bi[c, pl.ds(dd * 16, 16)] = 0.5 * (s * s - ss)
                    return ()

                lax.fori_loop(0, _CB, elem_body, ())
                pltpu.sync_copy(bi, out_hbm.at[pl.ds(b0, _CB)])
                return ()

            lax.fori_loop(0, NBH, block_body, ())
            return ()

        lax.fori_loop(0, NH, half_body, ())

    return sc_kernel(x_t, tab128)


_VC = 8192
_NVC = -(-_V // _VC)


def _tc_relayout(tab_t):
    def body(in_ref, out_ref):
        x = in_ref[0]
        xt = jnp.transpose(x)
        out_ref[0] = jnp.concatenate([xt, xt], axis=1)

    return pl.pallas_call(
        body,
        grid=(_F, _NVC),
        in_specs=[pl.BlockSpec((1, _D, _VC), lambda f, v: (f, 0, v))],
        out_specs=pl.BlockSpec((1, _VC, 2 * _D), lambda f, v: (f, v, 0)),
        out_shape=jax.ShapeDtypeStruct((_F, _V, 2 * _D), jnp.float32),
    )(tab_t)


def _tc_head(bi, x, W1, b1, W2, b2, W3t, b3, Wlt, bl):
    BT = 512
    grid = (_B // BT,)

    def body(bi_ref, x_ref, w1_ref, b1_ref, w2_ref, b2_ref, w3_ref,
             b3_ref, wl_ref, bl_ref, out_ref):
        h = jnp.dot(bi_ref[...], w1_ref[...],
                    preferred_element_type=jnp.float32) + b1_ref[...]
        h = jnp.maximum(h, 0.0)
        h = jnp.dot(h, w2_ref[...],
                    preferred_element_type=jnp.float32) + b2_ref[...]
        h = jnp.maximum(h, 0.0)
        deep = jnp.sum(h * w3_ref[...], axis=1, keepdims=True)
        xf = x_ref[...].astype(jnp.float32)
        lin = jnp.dot(xf, wl_ref[...])
        z = lin + deep + b3_ref[0, 0] + bl_ref[0, 0]
        out_ref[...] = jax.nn.sigmoid(z)

    return pl.pallas_call(
        body,
        grid=grid,
        in_specs=[
            pl.BlockSpec((BT, _D), lambda i: (i, 0)),
            pl.BlockSpec((BT, _F), lambda i: (i, 0)),
            pl.BlockSpec((_D, _H1), lambda i: (0, 0)),
            pl.BlockSpec((1, _H1), lambda i: (0, 0)),
            pl.BlockSpec((_H1, _H2), lambda i: (0, 0)),
            pl.BlockSpec((1, _H2), lambda i: (0, 0)),
            pl.BlockSpec((1, _H2), lambda i: (0, 0)),
            pl.BlockSpec((1, 1), lambda i: (0, 0)),
            pl.BlockSpec((_F, 1), lambda i: (0, 0)),
            pl.BlockSpec((1, 1), lambda i: (0, 0)),
        ],
        out_specs=pl.BlockSpec((BT, 1), lambda i: (i, 0)),
        out_shape=jax.ShapeDtypeStruct((_B, 1), jnp.float32),
    )(bi, x, W1, b1, W2, b2, W3t, b3, Wlt, bl)


def kernel(x, tables, Wl, bl, W1, b1, W2, b2, W3, b3):
    x = x.astype(jnp.int32)
    x_t = x.T
    tab_t = jnp.swapaxes(tables, 1, 2)
    tab128 = _tc_relayout(tab_t).reshape(_F * _V, 2 * _D)
    bi = _sc_bi_interaction(x_t, tab128)
    return _tc_head(
        bi, x, W1, b1.reshape(1, _H1), W2, b2.reshape(1, _H2),
        W3.reshape(1, _H2), b3.reshape(1, 1), Wl,
        bl.reshape(1, 1),
    )


# speedup vs baseline: 3.4624x; 1.0445x over previous
import functools

import jax
import jax.numpy as jnp
from jax import lax
from jax.experimental import pallas as pl
from jax.experimental.pallas import tpu as pltpu
from jax.experimental.pallas import tpu_sc as plsc

_B = 16384
_F = 26
_V = 100000
_D = 64
_H1 = 256
_H2 = 128

_NC = 2
_NS = 16
_NW = _NC * _NS
_BPW = _B // _NW
_CB = 32
_NBLK = _BPW // _CB


def _sc_bi_interaction(x_t, tab128):
    mesh = plsc.VectorSubcoreMesh(core_axis_name="c", subcore_axis_name="s")

    HSL = 256
    NH = _BPW // HSL
    NBH = HSL // _CB

    @functools.partial(
        pl.kernel,
        out_type=jax.ShapeDtypeStruct((_B, _D), jnp.float32),
        mesh=mesh,
        scratch_types=[
            pltpu.VMEM((_F, HSL), jnp.int32),
            pltpu.VMEM((_F, _CB), jnp.int32),
            pltpu.VMEM((_F, _CB, 2 * _D), jnp.float32),
            pltpu.VMEM((_CB, _D), jnp.float32),
            pltpu.SemaphoreType.DMA,
        ],
        compiler_params=pltpu.CompilerParams(use_tc_tiling_on_sc=True,
                                             needs_layout_passes=False),
    )
    def sc_kernel(x_hbm, tab_hbm, out_hbm, xw, idxb, rows, bi, sem):
        wid = lax.axis_index("s") * _NC + lax.axis_index("c")
        base = wid * _BPW

        def half_body(h, _):
            h0 = base + h * HSL
            pltpu.sync_copy(x_hbm.at[:, pl.ds(h0, HSL)], xw)

            def block_body(blk, _):
                lo = blk * _CB
                b0 = h0 + lo
                for f in range(_F):
                    for j in range(_CB // 16):
                        idxb[f, pl.ds(j * 16, 16)] = (
                            xw[f, pl.ds(lo + j * 16, 16)] + f * _V)
                copies = [
                    pltpu.async_copy(tab_hbm.at[idxb.at[f]], rows.at[f], sem)
                    for f in range(_F)
                ]
                for c in copies:
                    c.wait()

                def elem_body(c, _):
                    for dd in range(_D // 16):
                        s = jnp.zeros((16,), jnp.float32)
                        ss = jnp.zeros((16,), jnp.float32)
                        for f in range(_F):
                            r = rows[f, c, pl.ds(dd * 16, 16)]
                            s = s + r
                            ss = ss + r * r
                        bi[c, pl.ds(dd * 16, 16)] = 0.5 * (s * s - ss)
                    return ()

                lax.fori_loop(0, _CB, elem_body, ())
                pltpu.sync_copy(bi, out_hbm.at[pl.ds(b0, _CB)])
                return ()

            lax.fori_loop(0, NBH, block_body, ())
            return ()

        lax.fori_loop(0, NH, half_body, ())

    return sc_kernel(x_t, tab128)


_VC = 16384
_NVC = -(-_V // _VC)


def _tc_relayout(tab_t):
    def body(in_ref, out_ref):
        x = in_ref[0]
        xt = jnp.transpose(x)
        out_ref[0] = jnp.concatenate([xt, xt], axis=1)

    return pl.pallas_call(
        body,
        grid=(_F, _NVC),
        in_specs=[pl.BlockSpec((1, _D, _VC), lambda f, v: (f, 0, v))],
        out_specs=pl.BlockSpec((1, _VC, 2 * _D), lambda f, v: (f, v, 0)),
        out_shape=jax.ShapeDtypeStruct((_F, _V, 2 * _D), jnp.float32),
    )(tab_t)


def _tc_head(bi, x, W1, b1, W2, b2, W3t, b3, Wlt, bl):
    BT = 512
    grid = (_B // BT,)

    def body(bi_ref, x_ref, w1_ref, b1_ref, w2_ref, b2_ref, w3_ref,
             b3_ref, wl_ref, bl_ref, out_ref):
        h = jnp.dot(bi_ref[...], w1_ref[...],
                    preferred_element_type=jnp.float32) + b1_ref[...]
        h = jnp.maximum(h, 0.0)
        h = jnp.dot(h, w2_ref[...],
                    preferred_element_type=jnp.float32) + b2_ref[...]
        h = jnp.maximum(h, 0.0)
        deep = jnp.sum(h * w3_ref[...], axis=1, keepdims=True)
        xf = x_ref[...].astype(jnp.float32)
        lin = jnp.dot(xf, wl_ref[...])
        z = lin + deep + b3_ref[0, 0] + bl_ref[0, 0]
        out_ref[...] = jax.nn.sigmoid(z)

    return pl.pallas_call(
        body,
        grid=grid,
        in_specs=[
            pl.BlockSpec((BT, _D), lambda i: (i, 0)),
            pl.BlockSpec((BT, _F), lambda i: (i, 0)),
            pl.BlockSpec((_D, _H1), lambda i: (0, 0)),
            pl.BlockSpec((1, _H1), lambda i: (0, 0)),
            pl.BlockSpec((_H1, _H2), lambda i: (0, 0)),
            pl.BlockSpec((1, _H2), lambda i: (0, 0)),
            pl.BlockSpec((1, _H2), lambda i: (0, 0)),
            pl.BlockSpec((1, 1), lambda i: (0, 0)),
            pl.BlockSpec((_F, 1), lambda i: (0, 0)),
            pl.BlockSpec((1, 1), lambda i: (0, 0)),
        ],
        out_specs=pl.BlockSpec((BT, 1), lambda i: (i, 0)),
        out_shape=jax.ShapeDtypeStruct((_B, 1), jnp.float32),
    )(bi, x, W1, b1, W2, b2, W3t, b3, Wlt, bl)


def kernel(x, tables, Wl, bl, W1, b1, W2, b2, W3, b3):
    x = x.astype(jnp.int32)
    x_t = x.T
    tab_t = jnp.swapaxes(tables, 1, 2)
    tab128 = _tc_relayout(tab_t).reshape(_F * _V, 2 * _D)
    bi = _sc_bi_interaction(x_t, tab128)
    return _tc_head(
        bi, x, W1, b1.reshape(1, _H1), W2, b2.reshape(1, _H2),
        W3.reshape(1, _H2), b3.reshape(1, 1), Wl,
        bl.reshape(1, 1),
    )
